# baseline (speedup 1.0000x reference)
import functools

import jax
import jax.numpy as jnp
from jax import lax
from jax.experimental import pallas as pl
from jax.experimental.pallas import tpu as pltpu
from jax.experimental.pallas import tpu_sc as plsc

NC = 2
NS = 16
NW = NC * NS
CHUNK = 128
BN = 1000


def _sc_deg(dst4, n_nodes, e_pad):
    epw = e_pad // NW
    nch = epw // CHUNK
    depth = 8
    per_tile = ((n_nodes + 8 + NS * 128 - 1) // (NS * 128)) * 128
    acc_n = per_tile * NS
    mesh = plsc.VectorSubcoreMesh(core_axis_name="c", subcore_axis_name="s")

    @functools.partial(
        pl.kernel,
        mesh=mesh,
        out_type=jax.ShapeDtypeStruct((NC, 1, acc_n), jnp.float32),
        scratch_types=[
            pltpu.VMEM((nch, 1, CHUNK), jnp.int32),
            pltpu.VMEM((CHUNK,), jnp.float32),
            pltpu.VMEM((per_tile,), jnp.float32),
            pltpu.VMEM_SHARED((acc_n,), jnp.float32),
            pltpu.SemaphoreType.DMA,
        ],
    )
    def deg_kernel(dst_hbm, out_hbm, dsts, onesv, zb, acc, ssem):
        c = lax.axis_index("c")
        s = lax.axis_index("s")
        wid = s * NC + c
        pltpu.sync_copy(dst_hbm.at[wid], dsts)
        for k in range(CHUNK // 16):
            onesv[pl.ds(k * 16, 16)] = jnp.ones((16,), jnp.float32)

        def zfill(i, carry):
            zb[pl.ds(i * 16, 16)] = jnp.zeros((16,), jnp.float32)
            return carry

        lax.fori_loop(0, per_tile // 16, zfill, 0)
        pltpu.sync_copy(zb, acc.at[pl.ds(s * per_tile, per_tile)])
        plsc.subcore_barrier()

        for b in range(depth):
            pltpu.async_copy(onesv, acc.at[dsts.at[b, 0]], ssem, add=True)

        def body(j, carry):
            @pl.when(j + depth < nch)
            def _():
                pltpu.async_copy(onesv, acc.at[dsts.at[j + depth, 0]], ssem,
                                 add=True)

            pltpu.make_async_copy(onesv, acc.at[dsts.at[0, 0]], ssem).wait()
            return carry

        lax.fori_loop(0, nch, body, 0)
        plsc.subcore_barrier()
        pltpu.sync_copy(acc.at[pl.ds(s * per_tile, per_tile)],
                        out_hbm.at[c, 0, pl.ds(s * per_tile, per_tile)])

    return deg_kernel(dst4)


def _sc_agg(g, zeros_nd, src4, dst4, n_nodes, d, e_pad):
    epw = e_pad // NW
    nch = epw // CHUNK
    nblk = nch // 2
    rpt_a = ((n_nodes + NS - 1) // NS + 7) // 8 * 8
    rpt_l = n_nodes - (NS - 1) * rpt_a
    mesh = plsc.VectorSubcoreMesh(core_axis_name="c", subcore_axis_name="s")

    @functools.partial(
        pl.kernel,
        mesh=mesh,
        out_type=jax.ShapeDtypeStruct((NC, n_nodes, d), jnp.float32),
        scratch_types=[
            pltpu.VMEM((nblk, 1, CHUNK), jnp.int32),
            pltpu.VMEM((nblk, 1, CHUNK), jnp.int32),
            pltpu.VMEM((CHUNK, d), jnp.float32),
            pltpu.VMEM((CHUNK, d), jnp.float32),
            pltpu.VMEM_SHARED((n_nodes + 8, d), jnp.float32),
            pltpu.SemaphoreType.DMA,
            pltpu.SemaphoreType.DMA,
            pltpu.SemaphoreType.DMA,
        ],
    )
    def agg_kernel(g_hbm, z_hbm, src_hbm, dst_hbm, out_hbm,
                   srcs, dsts, rows0, rows1, acc, gsem0, gsem1, isem):
        c = lax.axis_index("c")
        s = lax.axis_index("s")
        wid = s * NC + c

        def init_spans(src, go):
            @pl.when(s < NS - 1)
            def _():
                go(src.at[pl.ds(s * rpt_a, rpt_a)],
                   acc.at[pl.ds(s * rpt_a, rpt_a)])

            @pl.when(s == NS - 1)
            def _():
                go(src.at[pl.ds((NS - 1) * rpt_a, rpt_l)],
                   acc.at[pl.ds((NS - 1) * rpt_a, rpt_l)])

        def start(a, b):
            pltpu.async_copy(a, b, isem)

        def drain(a, b):
            pltpu.make_async_copy(a, b, isem).wait()

        @pl.when(c == 0)
        def _():
            init_spans(g_hbm, start)

        @pl.when(c != 0)
        def _():
            init_spans(z_hbm, start)

        pltpu.sync_copy(src_hbm.at[wid, pl.ds(0, nblk)], srcs)
        pltpu.sync_copy(dst_hbm.at[wid, pl.ds(0, nblk)], dsts)
        pltpu.async_copy(g_hbm.at[srcs.at[0, 0]], rows0, gsem0)
        init_spans(g_hbm, drain)
        plsc.subcore_barrier()

        def inner(jj, carry2):
            j0 = 2 * jj
            j1 = j0 + 1
            pltpu.async_copy(g_hbm.at[srcs.at[j1, 0]], rows1, gsem1)
            pltpu.make_async_copy(g_hbm.at[pl.ds(0, CHUNK)], rows0,
                                  gsem0).wait()
            pltpu.sync_copy(rows0, acc.at[dsts.at[j0, 0]], add=True)

            @pl.when(j1 + 1 < nblk)
            def _():
                pltpu.async_copy(g_hbm.at[srcs.at[j1 + 1, 0]], rows0, gsem0)

            pltpu.make_async_copy(g_hbm.at[pl.ds(0, CHUNK)], rows1,
                                  gsem1).wait()
            pltpu.sync_copy(rows1, acc.at[dsts.at[j1, 0]], add=True)
            return carry2

        for k in range(nch // nblk):
            if k > 0:
                pltpu.sync_copy(src_hbm.at[wid, pl.ds(k * nblk, nblk)], srcs)
                pltpu.sync_copy(dst_hbm.at[wid, pl.ds(k * nblk, nblk)], dsts)
                pltpu.async_copy(g_hbm.at[srcs.at[0, 0]], rows0, gsem0)
            lax.fori_loop(0, nblk // 2, inner, 0)
        plsc.subcore_barrier()

        @pl.when(s < NS - 1)
        def _():
            pltpu.sync_copy(acc.at[pl.ds(s * rpt_a, rpt_a)],
                            out_hbm.at[c, pl.ds(s * rpt_a, rpt_a)])

        @pl.when(s == NS - 1)
        def _():
            pltpu.sync_copy(acc.at[pl.ds((NS - 1) * rpt_a, rpt_l)],
                            out_hbm.at[c, pl.ds((NS - 1) * rpt_a, rpt_l)])

    return agg_kernel(g, zeros_nd, src4, dst4)


def _tc_h0(x, w_in_t, b_in):
    n, d = x.shape

    def body(x_ref, wi_ref, bi_ref, h0_ref):
        h0_ref[...] = jnp.dot(x_ref[...], wi_ref[...],
                              preferred_element_type=jnp.float32) + bi_ref[...]

    return pl.pallas_call(
        body,
        grid=(n // BN,),
        in_specs=[
            pl.BlockSpec((BN, d), lambda i: (i, 0)),
            pl.BlockSpec((d, d), lambda i: (0, 0)),
            pl.BlockSpec((1, d), lambda i: (0, 0)),
        ],
        out_specs=pl.BlockSpec((BN, d), lambda i: (i, 0)),
        out_shape=jax.ShapeDtypeStruct((n, d), jnp.float32),
    )(x, w_in_t, b_in)


def _tc_g0(h0, w0_t, degp):
    n, d = h0.shape

    def body(h0_ref, w0_ref, dg_ref, dinv_ref, g0_ref):
        deg = dg_ref[0] + dg_ref[1] + 1.0
        dinv = lax.rsqrt(jnp.maximum(deg, 1e-12))
        dinv_ref[...] = dinv
        g0_ref[...] = jnp.dot(h0_ref[...], w0_ref[...],
                              preferred_element_type=jnp.float32) * dinv

    return pl.pallas_call(
        body,
        grid=(n // BN,),
        in_specs=[
            pl.BlockSpec((BN, d), lambda i: (i, 0)),
            pl.BlockSpec((d, d), lambda i: (0, 0)),
            pl.BlockSpec((2, BN, 1), lambda i: (0, i, 0)),
        ],
        out_specs=[
            pl.BlockSpec((BN, 1), lambda i: (i, 0)),
            pl.BlockSpec((BN, d), lambda i: (i, 0)),
        ],
        out_shape=[
            jax.ShapeDtypeStruct((n, 1), jnp.float32),
            jax.ShapeDtypeStruct((n, d), jnp.float32),
        ],
    )(h0, w0_t, degp)


def _tc_layer(p, dinv, b, h_prev, w_next_t):
    n, d = h_prev.shape

    def body(p_ref, dinv_ref, b_ref, h_ref, wn_ref, hn_ref, gn_ref):
        dinv = dinv_ref[...]
        hn = jnp.tanh(dinv * (p_ref[0] + p_ref[1]) + b_ref[...]) + h_ref[...]
        hn_ref[...] = hn
        gn_ref[...] = jnp.dot(hn, wn_ref[...],
                              preferred_element_type=jnp.float32) * dinv

    return pl.pallas_call(
        body,
        grid=(n // BN,),
        in_specs=[
            pl.BlockSpec((2, BN, d), lambda i: (0, i, 0)),
            pl.BlockSpec((BN, 1), lambda i: (i, 0)),
            pl.BlockSpec((1, d), lambda i: (0, 0)),
            pl.BlockSpec((BN, d), lambda i: (i, 0)),
            pl.BlockSpec((d, d), lambda i: (0, 0)),
        ],
        out_specs=[
            pl.BlockSpec((BN, d), lambda i: (i, 0)),
            pl.BlockSpec((BN, d), lambda i: (i, 0)),
        ],
        out_shape=[
            jax.ShapeDtypeStruct((n, d), jnp.float32),
            jax.ShapeDtypeStruct((n, d), jnp.float32),
        ],
    )(p, dinv, b, h_prev, w_next_t)


def _tc_final(p, dinv, b, h_prev, w_out_t, b_out):
    n, d = h_prev.shape

    def body(p_ref, dinv_ref, b_ref, h_ref, wo_ref, bo_ref, o_ref):
        hn = (jnp.tanh(dinv_ref[...] * (p_ref[0] + p_ref[1]) + b_ref[...])
              + h_ref[...])
        o = jnp.dot(hn, wo_ref[...],
                    preferred_element_type=jnp.float32) + bo_ref[...]
        sh = o - jnp.max(o, axis=1, keepdims=True)
        o_ref[...] = sh - jnp.log(jnp.sum(jnp.exp(sh), axis=1, keepdims=True))

    return pl.pallas_call(
        body,
        grid=(n // BN,),
        in_specs=[
            pl.BlockSpec((2, BN, d), lambda i: (0, i, 0)),
            pl.BlockSpec((BN, 1), lambda i: (i, 0)),
            pl.BlockSpec((1, d), lambda i: (0, 0)),
            pl.BlockSpec((BN, d), lambda i: (i, 0)),
            pl.BlockSpec((d, d), lambda i: (0, 0)),
            pl.BlockSpec((1, d), lambda i: (0, 0)),
        ],
        out_specs=pl.BlockSpec((BN, d), lambda i: (i, 0)),
        out_shape=jax.ShapeDtypeStruct((n, d), jnp.float32),
    )(p, dinv, b, h_prev, w_out_t, b_out)


def kernel(x, edge_index, W_in, b_in, W_convs, b_convs, W_out, b_out):
    n, d = x.shape
    e = edge_index.shape[1]
    num_layers = W_convs.shape[0]

    src = edge_index[0]
    dst = edge_index[1]
    e_pad = ((e + NW * CHUNK * 2 - 1) // (NW * CHUNK * 2)) * (NW * CHUNK * 2)
    npad = e_pad - e
    pad_ids = jnp.arange(npad, dtype=jnp.int32)
    src_p = jnp.concatenate([src, pad_ids % 64])
    dst_p = jnp.concatenate([dst, n + (pad_ids % 8)])
    nch = e_pad // NW // CHUNK
    src4 = src_p.reshape(NW, nch, 1, CHUNK)
    dst4 = dst_p.reshape(NW, nch, 1, CHUNK)

    zeros_nd = jnp.zeros((n, d), jnp.float32)
    b_in2 = b_in.reshape(1, d)
    b_out2 = b_out.reshape(1, d)

    deg_raw = _sc_deg(dst4, n, e_pad)
    degp = deg_raw[:, 0, :n].reshape(2, n, 1)

    h = _tc_h0(x, W_in.T, b_in2)
    dinv, g = _tc_g0(h, W_convs[0].T, degp)
    for i in range(num_layers):
        p = _sc_agg(g, zeros_nd, src4, dst4, n, d, e_pad)
        b_i = b_convs[i].reshape(1, d)
        if i + 1 < num_layers:
            h, g = _tc_layer(p, dinv, b_i, h, W_convs[i + 1].T)
        else:
            return _tc_final(p, dinv, b_i, h, W_out.T, b_out2)


# speedup vs baseline: 1.0355x; 1.0355x over previous
import functools

import jax
import jax.numpy as jnp
from jax import lax
from jax.experimental import pallas as pl
from jax.experimental.pallas import tpu as pltpu
from jax.experimental.pallas import tpu_sc as plsc

NC = 2
NS = 16
NW = NC * NS
CHUNK = 128
BN = 5000


def _sc_deg(dst4, n_nodes, e_pad):
    epw = e_pad // NW
    nch = epw // CHUNK
    depth = 8
    per_tile = ((n_nodes + 8 + NS * 128 - 1) // (NS * 128)) * 128
    acc_n = per_tile * NS
    mesh = plsc.VectorSubcoreMesh(core_axis_name="c", subcore_axis_name="s")

    @functools.partial(
        pl.kernel,
        mesh=mesh,
        out_type=jax.ShapeDtypeStruct((NC, 1, acc_n), jnp.float32),
        scratch_types=[
            pltpu.VMEM((nch, 1, CHUNK), jnp.int32),
            pltpu.VMEM((CHUNK,), jnp.float32),
            pltpu.VMEM((per_tile,), jnp.float32),
            pltpu.VMEM_SHARED((acc_n,), jnp.float32),
            pltpu.SemaphoreType.DMA,
        ],
    )
    def deg_kernel(dst_hbm, out_hbm, dsts, onesv, zb, acc, ssem):
        c = lax.axis_index("c")
        s = lax.axis_index("s")
        wid = s * NC + c
        pltpu.sync_copy(dst_hbm.at[wid], dsts)
        for k in range(CHUNK // 16):
            onesv[pl.ds(k * 16, 16)] = jnp.ones((16,), jnp.float32)

        def zfill(i, carry):
            zb[pl.ds(i * 16, 16)] = jnp.zeros((16,), jnp.float32)
            return carry

        lax.fori_loop(0, per_tile // 16, zfill, 0)
        pltpu.sync_copy(zb, acc.at[pl.ds(s * per_tile, per_tile)])
        plsc.subcore_barrier()

        for b in range(depth):
            pltpu.async_copy(onesv, acc.at[dsts.at[b, 0]], ssem, add=True)

        def body(j, carry):
            @pl.when(j + depth < nch)
            def _():
                pltpu.async_copy(onesv, acc.at[dsts.at[j + depth, 0]], ssem,
                                 add=True)

            pltpu.make_async_copy(onesv, acc.at[dsts.at[0, 0]], ssem).wait()
            return carry

        lax.fori_loop(0, nch, body, 0)
        plsc.subcore_barrier()
        pltpu.sync_copy(acc.at[pl.ds(s * per_tile, per_tile)],
                        out_hbm.at[c, 0, pl.ds(s * per_tile, per_tile)])

    return deg_kernel(dst4)


def _sc_agg(g, zeros_nd, src4, dst4, n_nodes, d, e_pad):
    epw = e_pad // NW
    nch = epw // CHUNK
    nblk = nch // 2
    rpt_a = ((n_nodes + NS - 1) // NS + 7) // 8 * 8
    rpt_l = n_nodes - (NS - 1) * rpt_a
    mesh = plsc.VectorSubcoreMesh(core_axis_name="c", subcore_axis_name="s")

    @functools.partial(
        pl.kernel,
        mesh=mesh,
        out_type=jax.ShapeDtypeStruct((NC, n_nodes, d), jnp.float32),
        scratch_types=[
            pltpu.VMEM((nblk, 1, CHUNK), jnp.int32),
            pltpu.VMEM((nblk, 1, CHUNK), jnp.int32),
            pltpu.VMEM((CHUNK, d), jnp.float32),
            pltpu.VMEM((CHUNK, d), jnp.float32),
            pltpu.VMEM_SHARED((n_nodes + 8, d), jnp.float32),
            pltpu.SemaphoreType.DMA,
            pltpu.SemaphoreType.DMA,
            pltpu.SemaphoreType.DMA,
        ],
    )
    def agg_kernel(g_hbm, z_hbm, src_hbm, dst_hbm, out_hbm,
                   srcs, dsts, rows0, rows1, acc, gsem0, gsem1, isem):
        c = lax.axis_index("c")
        s = lax.axis_index("s")
        wid = s * NC + c

        def init_spans(src, go):
            @pl.when(s < NS - 1)
            def _():
                go(src.at[pl.ds(s * rpt_a, rpt_a)],
                   acc.at[pl.ds(s * rpt_a, rpt_a)])

            @pl.when(s == NS - 1)
            def _():
                go(src.at[pl.ds((NS - 1) * rpt_a, rpt_l)],
                   acc.at[pl.ds((NS - 1) * rpt_a, rpt_l)])

        def start(a, b):
            pltpu.async_copy(a, b, isem)

        def drain(a, b):
            pltpu.make_async_copy(a, b, isem).wait()

        @pl.when(c == 0)
        def _():
            init_spans(g_hbm, start)

        @pl.when(c != 0)
        def _():
            init_spans(z_hbm, start)

        pltpu.sync_copy(src_hbm.at[wid, pl.ds(0, nblk)], srcs)
        pltpu.sync_copy(dst_hbm.at[wid, pl.ds(0, nblk)], dsts)
        pltpu.async_copy(g_hbm.at[srcs.at[0, 0]], rows0, gsem0)
        init_spans(g_hbm, drain)
        plsc.subcore_barrier()

        def inner(jj, carry2):
            j0 = 2 * jj
            j1 = j0 + 1
            pltpu.async_copy(g_hbm.at[srcs.at[j1, 0]], rows1, gsem1)
            pltpu.make_async_copy(g_hbm.at[pl.ds(0, CHUNK)], rows0,
                                  gsem0).wait()
            pltpu.sync_copy(rows0, acc.at[dsts.at[j0, 0]], add=True)

            @pl.when(j1 + 1 < nblk)
            def _():
                pltpu.async_copy(g_hbm.at[srcs.at[j1 + 1, 0]], rows0, gsem0)

            pltpu.make_async_copy(g_hbm.at[pl.ds(0, CHUNK)], rows1,
                                  gsem1).wait()
            pltpu.sync_copy(rows1, acc.at[dsts.at[j1, 0]], add=True)
            return carry2

        for k in range(nch // nblk):
            if k > 0:
                pltpu.sync_copy(src_hbm.at[wid, pl.ds(k * nblk, nblk)], srcs)
                pltpu.sync_copy(dst_hbm.at[wid, pl.ds(k * nblk, nblk)], dsts)
                pltpu.async_copy(g_hbm.at[srcs.at[0, 0]], rows0, gsem0)
            lax.fori_loop(0, nblk // 2, inner, 0)
        plsc.subcore_barrier()

        @pl.when(s < NS - 1)
        def _():
            pltpu.sync_copy(acc.at[pl.ds(s * rpt_a, rpt_a)],
                            out_hbm.at[c, pl.ds(s * rpt_a, rpt_a)])

        @pl.when(s == NS - 1)
        def _():
            pltpu.sync_copy(acc.at[pl.ds((NS - 1) * rpt_a, rpt_l)],
                            out_hbm.at[c, pl.ds((NS - 1) * rpt_a, rpt_l)])

    return agg_kernel(g, zeros_nd, src4, dst4)


def _tc_h0(x, w_in_t, b_in):
    n, d = x.shape

    def body(x_ref, wi_ref, bi_ref, h0_ref):
        h0_ref[...] = jnp.dot(x_ref[...], wi_ref[...],
                              preferred_element_type=jnp.float32) + bi_ref[...]

    return pl.pallas_call(
        body,
        grid=(n // BN,),
        in_specs=[
            pl.BlockSpec((BN, d), lambda i: (i, 0)),
            pl.BlockSpec((d, d), lambda i: (0, 0)),
            pl.BlockSpec((1, d), lambda i: (0, 0)),
        ],
        out_specs=pl.BlockSpec((BN, d), lambda i: (i, 0)),
        out_shape=jax.ShapeDtypeStruct((n, d), jnp.float32),
    )(x, w_in_t, b_in)


def _tc_g0(h0, w0_t, degp):
    n, d = h0.shape

    def body(h0_ref, w0_ref, dg_ref, dinv_ref, g0_ref):
        deg = dg_ref[0] + dg_ref[1] + 1.0
        dinv = lax.rsqrt(jnp.maximum(deg, 1e-12))
        dinv_ref[...] = dinv
        g0_ref[...] = jnp.dot(h0_ref[...], w0_ref[...],
                              preferred_element_type=jnp.float32) * dinv

    return pl.pallas_call(
        body,
        grid=(n // BN,),
        in_specs=[
            pl.BlockSpec((BN, d), lambda i: (i, 0)),
            pl.BlockSpec((d, d), lambda i: (0, 0)),
            pl.BlockSpec((2, BN, 1), lambda i: (0, i, 0)),
        ],
        out_specs=[
            pl.BlockSpec((BN, 1), lambda i: (i, 0)),
            pl.BlockSpec((BN, d), lambda i: (i, 0)),
        ],
        out_shape=[
            jax.ShapeDtypeStruct((n, 1), jnp.float32),
            jax.ShapeDtypeStruct((n, d), jnp.float32),
        ],
    )(h0, w0_t, degp)


def _tc_layer(p, dinv, b, h_prev, w_next_t):
    n, d = h_prev.shape

    def body(p_ref, dinv_ref, b_ref, h_ref, wn_ref, hn_ref, gn_ref):
        dinv = dinv_ref[...]
        hn = jnp.tanh(dinv * (p_ref[0] + p_ref[1]) + b_ref[...]) + h_ref[...]
        hn_ref[...] = hn
        gn_ref[...] = jnp.dot(hn, wn_ref[...],
                              preferred_element_type=jnp.float32) * dinv

    return pl.pallas_call(
        body,
        grid=(n // BN,),
        in_specs=[
            pl.BlockSpec((2, BN, d), lambda i: (0, i, 0)),
            pl.BlockSpec((BN, 1), lambda i: (i, 0)),
            pl.BlockSpec((1, d), lambda i: (0, 0)),
            pl.BlockSpec((BN, d), lambda i: (i, 0)),
            pl.BlockSpec((d, d), lambda i: (0, 0)),
        ],
        out_specs=[
            pl.BlockSpec((BN, d), lambda i: (i, 0)),
            pl.BlockSpec((BN, d), lambda i: (i, 0)),
        ],
        out_shape=[
            jax.ShapeDtypeStruct((n, d), jnp.float32),
            jax.ShapeDtypeStruct((n, d), jnp.float32),
        ],
    )(p, dinv, b, h_prev, w_next_t)


def _tc_final(p, dinv, b, h_prev, w_out_t, b_out):
    n, d = h_prev.shape

    def body(p_ref, dinv_ref, b_ref, h_ref, wo_ref, bo_ref, o_ref):
        hn = (jnp.tanh(dinv_ref[...] * (p_ref[0] + p_ref[1]) + b_ref[...])
              + h_ref[...])
        o = jnp.dot(hn, wo_ref[...],
                    preferred_element_type=jnp.float32) + bo_ref[...]
        sh = o - jnp.max(o, axis=1, keepdims=True)
        o_ref[...] = sh - jnp.log(jnp.sum(jnp.exp(sh), axis=1, keepdims=True))

    return pl.pallas_call(
        body,
        grid=(n // BN,),
        in_specs=[
            pl.BlockSpec((2, BN, d), lambda i: (0, i, 0)),
            pl.BlockSpec((BN, 1), lambda i: (i, 0)),
            pl.BlockSpec((1, d), lambda i: (0, 0)),
            pl.BlockSpec((BN, d), lambda i: (i, 0)),
            pl.BlockSpec((d, d), lambda i: (0, 0)),
            pl.BlockSpec((1, d), lambda i: (0, 0)),
        ],
        out_specs=pl.BlockSpec((BN, d), lambda i: (i, 0)),
        out_shape=jax.ShapeDtypeStruct((n, d), jnp.float32),
    )(p, dinv, b, h_prev, w_out_t, b_out)


def kernel(x, edge_index, W_in, b_in, W_convs, b_convs, W_out, b_out):
    n, d = x.shape
    e = edge_index.shape[1]
    num_layers = W_convs.shape[0]

    src = edge_index[0]
    dst = edge_index[1]
    e_pad = ((e + NW * CHUNK * 2 - 1) // (NW * CHUNK * 2)) * (NW * CHUNK * 2)
    npad = e_pad - e
    pad_ids = jnp.arange(npad, dtype=jnp.int32)
    src_p = jnp.concatenate([src, pad_ids % 64])
    dst_p = jnp.concatenate([dst, n + (pad_ids % 8)])
    nch = e_pad // NW // CHUNK
    src4 = src_p.reshape(NW, nch, 1, CHUNK)
    dst4 = dst_p.reshape(NW, nch, 1, CHUNK)

    zeros_nd = jnp.zeros((n, d), jnp.float32)
    b_in2 = b_in.reshape(1, d)
    b_out2 = b_out.reshape(1, d)

    deg_raw = _sc_deg(dst4, n, e_pad)
    degp = deg_raw[:, 0, :n].reshape(2, n, 1)

    h = _tc_h0(x, W_in.T, b_in2)
    dinv, g = _tc_g0(h, W_convs[0].T, degp)
    for i in range(num_layers):
        p = _sc_agg(g, zeros_nd, src4, dst4, n, d, e_pad)
        b_i = b_convs[i].reshape(1, d)
        if i + 1 < num_layers:
            h, g = _tc_layer(p, dinv, b_i, h, W_convs[i + 1].T)
        else:
            return _tc_final(p, dinv, b_i, h, W_out.T, b_out2)
